# baseline (speedup 1.0000x reference)
import functools

import jax
import jax.numpy as jnp
from jax import lax
from jax.experimental import pallas as pl
from jax.experimental.pallas import tpu as pltpu
from jax.experimental.pallas import tpu_sc as plsc

N = 10000
E = 320000
D = 128

NC = 2
NS = 16
NW = NC * NS
EPT = E // NW

WH = 2000
WE = 200
F = 16
NP = D // F
RPT = N // NS

_MESH = plsc.VectorSubcoreMesh(core_axis_name="c", subcore_axis_name="s")


def _hist_body(row_hbm, zeros_hbm, degp_hbm, roww, onesw, acc):
    c = lax.axis_index("c")
    s = lax.axis_index("s")
    wid = s * NC + c

    @pl.when(s == 0)
    def _():
        pltpu.sync_copy(zeros_hbm, acc)

    def fill(i, carry):
        onesw[pl.ds(16 * i, 16)] = jnp.ones((16,), jnp.float32)
        return carry

    lax.fori_loop(0, WH // 16, fill, 0)
    plsc.subcore_barrier()

    base = wid * EPT

    def window(w, carry):
        pltpu.sync_copy(row_hbm.at[pl.ds(base + w * WH, WH)], roww)
        pltpu.sync_copy(onesw, acc.at[roww], add=True)
        return carry

    lax.fori_loop(0, EPT // WH, window, 0)
    plsc.subcore_barrier()

    @pl.when(s == 0)
    def _():
        pltpu.sync_copy(acc, degp_hbm.at[c])


_SC_PARAMS = pltpu.CompilerParams(use_tc_tiling_on_sc=False,
                                  needs_layout_passes=False)

_hist = pl.kernel(
    _hist_body,
    out_type=jax.ShapeDtypeStruct((NC, N), jnp.float32),
    mesh=_MESH,
    compiler_params=_SC_PARAMS,
    scratch_types=[
        pltpu.VMEM((WH,), jnp.int32),
        pltpu.VMEM((WH,), jnp.float32),
        pltpu.VMEM_SHARED((N,), jnp.float32),
    ],
)


NWIN = EPT // WE


def _edge_body(row_hbm, col_hbm, ea_hbm, xcat_hbm, dis1_hbm,
               splat_hbm, out_hbm,
               idxall, drall, splix, wvec, eaw0, eaw1, g010, g011, m010, m011,
               acc01_sh, dis1_sh,
               sea0, sea1, sg0, sg1, ssc0, ssc1):
    c = lax.axis_index("c")
    s = lax.axis_index("s")
    wid = s * NC + c
    r0 = s * RPT
    base = wid * EPT

    eaw = (eaw0, eaw1)
    g01 = (g010, g011)
    m01 = (m010, m011)
    sea = (sea0, sea1)
    sg = (sg0, sg1)
    ssc = (ssc0, ssc1)

    pltpu.sync_copy(row_hbm.at[pl.ds(base, EPT)], idxall.at[0])
    pltpu.sync_copy(col_hbm.at[pl.ds(base, EPT)], idxall.at[1])

    pltpu.sync_copy(splat_hbm, splix)

    @pl.when(s == 0)
    def _():
        pltpu.sync_copy(dis1_hbm, dis1_sh)

    plsc.subcore_barrier()
    pltpu.sync_copy(dis1_sh.at[idxall.at[0]], drall)

    for p in range(NP):
        slab = pl.ds(p * F, F)

        @plsc.parallel_loop(0, WE, unroll=8)
        def _zfill(e):
            m010[e, pl.ds(0, 16)] = jnp.zeros((16,), jnp.float32)
            m010[e, pl.ds(16, 16)] = jnp.zeros((16,), jnp.float32)

        wvec[0, :] = jnp.zeros((16,), jnp.int32)
        for z in range(RPT // WE):
            pltpu.sync_copy(m010, acc01_sh.at[pl.ds(r0 + z * WE, WE), :])
        pltpu.sync_copy(m010.at[pl.ds(0, RPT % WE), :],
                        acc01_sh.at[pl.ds(r0 + (RPT // WE) * WE, RPT % WE), :])
        plsc.subcore_barrier()

        def _in_descs(w, b):
            e0 = w * WE
            return (
                pltpu.make_async_copy(
                    ea_hbm.at[pl.ds(base + e0, WE), slab], eaw[b], sea[b]),
                pltpu.make_async_copy(
                    xcat_hbm.at[p].at[idxall.at[0, pl.ds(e0, WE)]],
                    g01[b], sg[b]),
            )

        def _sc_desc(w, b):
            e0 = w * WE
            return pltpu.make_async_copy(
                m01[b], acc01_sh.at[idxall.at[1, pl.ds(e0, WE)]], ssc[b])

        def _compute(b, w0):
            eab, gb, mb = eaw[b], g01[b], m01[b]
            wv = wvec[0, :]

            @plsc.parallel_loop(0, WE, unroll=8)
            def _body(e):
                idxabs = splix[e, :] + wv
                dr = plsc.load_gather(drall, [idxabs])
                ead = eab[e, pl.ds(0, 16)] * dr
                mb[e, pl.ds(0, 16)] = jnp.maximum(
                    gb[e, pl.ds(0, 16)] + ead, 0.0)
                mb[e, pl.ds(16, 16)] = jnp.maximum(
                    gb[e, pl.ds(16, 16)] + ead, 0.0)

        for d in _in_descs(0, 0):
            d.start()

        def pair(wo, carry):
            for b in (0, 1):
                w = 2 * wo + b
                for d in _in_descs(w, b):
                    d.wait()

                @pl.when(w + 1 < NWIN)
                def _():
                    for d in _in_descs(w + 1, 1 - b):
                        d.start()

                @pl.when(wo >= 1)
                def _():
                    _sc_desc(w - 2, b).wait()

                _compute(b, w)
                wvec[0, :] = wvec[0, :] + WE
                _sc_desc(w, b).start(add=True)
            return carry

        lax.fori_loop(0, NWIN // 2, pair, 0)

        _sc_desc(NWIN - 2, 0).wait()
        _sc_desc(NWIN - 1, 1).wait()
        plsc.subcore_barrier()

        pltpu.sync_copy(acc01_sh.at[pl.ds(r0, RPT), pl.ds(0, 16)],
                        out_hbm.at[0, c, pl.ds(r0, RPT), slab])
        pltpu.sync_copy(acc01_sh.at[pl.ds(r0, RPT), pl.ds(16, 16)],
                        out_hbm.at[1, c, pl.ds(r0, RPT), slab])


_edge = pl.kernel(
    _edge_body,
    out_type=jax.ShapeDtypeStruct((2, NC, N, D), jnp.float32),
    mesh=_MESH,
    compiler_params=_SC_PARAMS,
    scratch_types=[
        pltpu.VMEM((2, EPT), jnp.int32),
        pltpu.VMEM((EPT,), jnp.float32),
        pltpu.VMEM((WE, 16), jnp.int32),
        pltpu.VMEM((1, 16), jnp.int32),
        pltpu.VMEM((WE, F), jnp.float32),
        pltpu.VMEM((WE, F), jnp.float32),
        pltpu.VMEM((WE, 32), jnp.float32),
        pltpu.VMEM((WE, 32), jnp.float32),
        pltpu.VMEM((WE, 32), jnp.float32),
        pltpu.VMEM((WE, 32), jnp.float32),
        pltpu.VMEM_SHARED((N, 32), jnp.float32),
        pltpu.VMEM_SHARED((N,), jnp.float32),
        pltpu.SemaphoreType.DMA,
        pltpu.SemaphoreType.DMA,
        pltpu.SemaphoreType.DMA,
        pltpu.SemaphoreType.DMA,
        pltpu.SemaphoreType.DMA,
        pltpu.SemaphoreType.DMA,
    ],
)


def _dense_body(degp_ref, s0_ref, s1_ref,
                wp0_ref, bp0_ref, g0_ref, be0_ref, wg0_ref, bg0_ref,
                wp1_ref, bp1_ref, g1_ref, be1_ref, wg1_ref, bg1_ref,
                xd0_ref, xd1_ref, deg_ref, dis_ref, disr_ref):
    deg = degp_ref[0] + degp_ref[1] + 1.0
    dis = lax.rsqrt(deg)
    deg_ref[...] = deg
    dis_ref[...] = dis
    disr_ref[...] = jnp.broadcast_to(dis, (N, 16))

    dn = (((1,), (1,)), ((), ()))

    def branch(s_ref, wp_ref, bp_ref, g_ref, be_ref, wg_ref, bg_ref, xd_ref):
        p = lax.dot_general(s_ref[...], wp_ref[...], dn,
                            preferred_element_type=jnp.float32)
        p = p + bp_ref[...]
        m = jnp.mean(p, axis=0, keepdims=True)
        v = jnp.mean((p - m) ** 2, axis=0, keepdims=True)
        n = (p - m) * lax.rsqrt(v + 1e-5) * g_ref[...] + be_ref[...]
        xin = jnp.maximum(n, 0.0)
        xs = lax.dot_general(xin, wg_ref[...], dn,
                             preferred_element_type=jnp.float32)
        xd_ref[...] = (xs + bg_ref[...]) * dis

    branch(s0_ref, wp0_ref, bp0_ref, g0_ref, be0_ref, wg0_ref, bg0_ref,
           xd0_ref)
    branch(s1_ref, wp1_ref, bp1_ref, g1_ref, be1_ref, wg1_ref, bg1_ref,
           xd1_ref)


_dense = pl.pallas_call(
    _dense_body,
    out_shape=(
        jax.ShapeDtypeStruct((N, D), jnp.float32),
        jax.ShapeDtypeStruct((N, D), jnp.float32),
        jax.ShapeDtypeStruct((N, 1), jnp.float32),
        jax.ShapeDtypeStruct((N, 1), jnp.float32),
        jax.ShapeDtypeStruct((N, 16), jnp.float32),
    ),
)


_RB = 2000
_NB = N // _RB


def _final_body(agg_ref, xd0_ref, xd1_ref, deg_ref, dis_ref, root_ref,
                g_ref, be_ref, out_ref, ys_ref, sums_ref, sqs_ref):
    ph = pl.program_id(0)
    rb = pl.program_id(1)

    @pl.when(jnp.logical_and(ph == 0, rb == 0))
    def _():
        sums_ref[...] = jnp.zeros_like(sums_ref)
        sqs_ref[...] = jnp.zeros_like(sqs_ref)

    rows = pl.ds(rb * _RB, _RB)

    @pl.when(ph == 0)
    def _():
        deg = deg_ref[...]
        sdeg = jnp.sqrt(deg)
        dis = dis_ref[...]
        for b in range(2):
            agg = agg_ref[b, 0] + agg_ref[b, 1]
            xd = xd0_ref[...] if b == 0 else xd1_ref[...]
            x = xd * sdeg
            self_t = jnp.maximum(x + root_ref[b], 0.0) / deg
            y = dis * agg + self_t
            ys_ref[b, rows, :] = y
            sums_ref[b, :] += jnp.sum(y, axis=0)
            sqs_ref[b, :] += jnp.sum(y * y, axis=0)
        out_ref[...] = jnp.zeros_like(out_ref)

    @pl.when(ph == 1)
    def _():
        acc = jnp.zeros((_RB, D), jnp.float32)
        for b in range(2):
            mean = sums_ref[b, :] / N
            var = sqs_ref[b, :] / N - mean * mean
            y = ys_ref[b, rows, :]
            st = (y - mean) * lax.rsqrt(var + 1e-5) * g_ref[b] + be_ref[b]
            acc = acc + st
        out_ref[...] = acc


_final = pl.pallas_call(
    _final_body,
    grid=(2, _NB),
    in_specs=[
        pl.BlockSpec((2, NC, _RB, D), lambda ph, rb: (0, 0, rb * (1 - ph), 0)),
        pl.BlockSpec((_RB, D), lambda ph, rb: (rb * (1 - ph), 0)),
        pl.BlockSpec((_RB, D), lambda ph, rb: (rb * (1 - ph), 0)),
        pl.BlockSpec((_RB, 1), lambda ph, rb: (rb, 0)),
        pl.BlockSpec((_RB, 1), lambda ph, rb: (rb, 0)),
        pl.BlockSpec((2, 1, D), lambda ph, rb: (0, 0, 0)),
        pl.BlockSpec((2, D), lambda ph, rb: (0, 0)),
        pl.BlockSpec((2, D), lambda ph, rb: (0, 0)),
    ],
    out_specs=pl.BlockSpec((_RB, D), lambda ph, rb: (rb, 0)),
    out_shape=jax.ShapeDtypeStruct((N, D), jnp.float32),
    scratch_shapes=[
        pltpu.VMEM((2, N, D), jnp.float32),
        pltpu.VMEM((2, D), jnp.float32),
        pltpu.VMEM((2, D), jnp.float32),
    ],
)


def kernel(s0, s1, edge_index, edge_attr, in_degree, out_degree, mat, batch,
           max_node, W_pre0, b_pre0, g_pre0, be_pre0, W_pre1, b_pre1, g_pre1,
           be_pre1, W_gcn0, b_gcn0, root0, g_op0, be_op0, W_gcn1, b_gcn1,
           root1, g_op1, be_op1):
    row = edge_index[0]
    col = edge_index[1]
    zeros_n = jnp.zeros((N,), jnp.float32)

    degp = _hist(row, zeros_n)
    xd0, xd1, deg, dis, disr = _dense(
        degp.reshape(NC, N, 1), s0, s1,
        W_pre0, b_pre0, g_pre0, be_pre0, W_gcn0, b_gcn0,
        W_pre1, b_pre1, g_pre1, be_pre1, W_gcn1, b_gcn1)

    splat = jnp.broadcast_to(
        jnp.arange(WE, dtype=jnp.int32)[:, None], (WE, 16))
    xcat = jnp.concatenate(
        [xd0.reshape(N, NP, F), xd1.reshape(N, NP, F)], axis=2)
    xcat = jnp.transpose(xcat, (1, 0, 2))
    agg = _edge(row, col, edge_attr, xcat, dis.reshape(N), splat)

    root = jnp.stack([root0, root1])
    g = jnp.stack([g_op0, g_op1])
    be = jnp.stack([be_op0, be_op1])
    return _final(agg, xd0, xd1, deg, dis, root, g, be)


# speedup vs baseline: 1.1416x; 1.1416x over previous
import functools

import jax
import jax.numpy as jnp
from jax import lax
from jax.experimental import pallas as pl
from jax.experimental.pallas import tpu as pltpu
from jax.experimental.pallas import tpu_sc as plsc

N = 10000
E = 320000
D = 128

NC = 2
NS = 16
NW = NC * NS
EPT = E // NW

WH = 2000
WE = 200
F = 16
NP = D // F
RPT = N // NS

_MESH = plsc.VectorSubcoreMesh(core_axis_name="c", subcore_axis_name="s")


def _hist_body(row_hbm, zeros_hbm, degp_hbm, roww, onesw, acc):
    c = lax.axis_index("c")
    s = lax.axis_index("s")
    wid = s * NC + c

    @pl.when(s == 0)
    def _():
        pltpu.sync_copy(zeros_hbm, acc)

    def fill(i, carry):
        onesw[pl.ds(16 * i, 16)] = jnp.ones((16,), jnp.float32)
        return carry

    lax.fori_loop(0, WH // 16, fill, 0)
    plsc.subcore_barrier()

    base = wid * EPT

    def window(w, carry):
        pltpu.sync_copy(row_hbm.at[pl.ds(base + w * WH, WH)], roww)
        pltpu.sync_copy(onesw, acc.at[roww], add=True)
        return carry

    lax.fori_loop(0, EPT // WH, window, 0)
    plsc.subcore_barrier()

    @pl.when(s == 0)
    def _():
        pltpu.sync_copy(acc, degp_hbm.at[c])


_SC_PARAMS = pltpu.CompilerParams(use_tc_tiling_on_sc=False,
                                  needs_layout_passes=False)

_hist = pl.kernel(
    _hist_body,
    out_type=jax.ShapeDtypeStruct((NC, N), jnp.float32),
    mesh=_MESH,
    compiler_params=_SC_PARAMS,
    scratch_types=[
        pltpu.VMEM((WH,), jnp.int32),
        pltpu.VMEM((WH,), jnp.float32),
        pltpu.VMEM_SHARED((N,), jnp.float32),
    ],
)


NWIN = EPT // WE


def _edge_body(row_hbm, col_hbm, ea_hbm, xd0_hbm, xd1_hbm, dis1_hbm,
               splat_hbm, out_hbm,
               idxall, drall, splix, wvec, eaw0, eaw1, g010, g011, m010, m011,
               xtab_sh, acc01_sh, dis1_sh,
               sea0, sea1, sg0, sg1, ssc0, ssc1):
    c = lax.axis_index("c")
    s = lax.axis_index("s")
    wid = s * NC + c
    r0 = s * RPT
    base = wid * EPT

    eaw = (eaw0, eaw1)
    g01 = (g010, g011)
    m01 = (m010, m011)
    sea = (sea0, sea1)
    sg = (sg0, sg1)
    ssc = (ssc0, ssc1)

    pltpu.sync_copy(row_hbm.at[pl.ds(base, EPT)], idxall.at[0])
    pltpu.sync_copy(col_hbm.at[pl.ds(base, EPT)], idxall.at[1])

    pltpu.sync_copy(splat_hbm, splix)

    @pl.when(s == 0)
    def _():
        pltpu.sync_copy(dis1_hbm, dis1_sh)

    plsc.subcore_barrier()
    pltpu.sync_copy(dis1_sh.at[idxall.at[0]], drall)

    for p in range(NP):
        slab = pl.ds(p * F, F)

        @pl.when(s == 0)
        def _():
            pltpu.sync_copy(xd0_hbm.at[:, slab], xtab_sh.at[:, pl.ds(0, 16)])
            pltpu.sync_copy(xd1_hbm.at[:, slab], xtab_sh.at[:, pl.ds(16, 16)])

        @plsc.parallel_loop(0, WE, unroll=8)
        def _zfill(e):
            m010[e, pl.ds(0, 16)] = jnp.zeros((16,), jnp.float32)
            m010[e, pl.ds(16, 16)] = jnp.zeros((16,), jnp.float32)

        wvec[0, :] = jnp.zeros((16,), jnp.int32)
        for z in range(RPT // WE):
            pltpu.sync_copy(m010, acc01_sh.at[pl.ds(r0 + z * WE, WE), :])
        pltpu.sync_copy(m010.at[pl.ds(0, RPT % WE), :],
                        acc01_sh.at[pl.ds(r0 + (RPT // WE) * WE, RPT % WE), :])
        plsc.subcore_barrier()

        def _in_descs(w, b):
            e0 = w * WE
            return (
                pltpu.make_async_copy(
                    ea_hbm.at[pl.ds(base + e0, WE), slab], eaw[b], sea[b]),
                pltpu.make_async_copy(
                    xtab_sh.at[idxall.at[0, pl.ds(e0, WE)]], g01[b], sg[b]),
            )

        def _sc_desc(w, b):
            e0 = w * WE
            return pltpu.make_async_copy(
                m01[b], acc01_sh.at[idxall.at[1, pl.ds(e0, WE)]], ssc[b])

        def _compute(b, w0):
            eab, gb, mb = eaw[b], g01[b], m01[b]
            wv = wvec[0, :]

            @plsc.parallel_loop(0, WE, unroll=8)
            def _body(e):
                idxabs = splix[e, :] + wv
                dr = plsc.load_gather(drall, [idxabs])
                ead = eab[e, pl.ds(0, 16)] * dr
                mb[e, pl.ds(0, 16)] = jnp.maximum(
                    gb[e, pl.ds(0, 16)] + ead, 0.0)
                mb[e, pl.ds(16, 16)] = jnp.maximum(
                    gb[e, pl.ds(16, 16)] + ead, 0.0)

        for d in _in_descs(0, 0):
            d.start()

        def pair(wo, carry):
            for b in (0, 1):
                w = 2 * wo + b
                for d in _in_descs(w, b):
                    d.wait()

                @pl.when(w + 1 < NWIN)
                def _():
                    for d in _in_descs(w + 1, 1 - b):
                        d.start()

                @pl.when(wo >= 1)
                def _():
                    _sc_desc(w - 2, b).wait()

                _compute(b, w)
                wvec[0, :] = wvec[0, :] + WE
                _sc_desc(w, b).start(add=True)
            return carry

        lax.fori_loop(0, NWIN // 2, pair, 0)

        _sc_desc(NWIN - 2, 0).wait()
        _sc_desc(NWIN - 1, 1).wait()
        plsc.subcore_barrier()

        pltpu.sync_copy(acc01_sh.at[pl.ds(r0, RPT), pl.ds(0, 16)],
                        out_hbm.at[0, c, pl.ds(r0, RPT), slab])
        pltpu.sync_copy(acc01_sh.at[pl.ds(r0, RPT), pl.ds(16, 16)],
                        out_hbm.at[1, c, pl.ds(r0, RPT), slab])


_edge = pl.kernel(
    _edge_body,
    out_type=jax.ShapeDtypeStruct((2, NC, N, D), jnp.float32),
    mesh=_MESH,
    compiler_params=_SC_PARAMS,
    scratch_types=[
        pltpu.VMEM((2, EPT), jnp.int32),
        pltpu.VMEM((EPT,), jnp.float32),
        pltpu.VMEM((WE, 16), jnp.int32),
        pltpu.VMEM((1, 16), jnp.int32),
        pltpu.VMEM((WE, F), jnp.float32),
        pltpu.VMEM((WE, F), jnp.float32),
        pltpu.VMEM((WE, 32), jnp.float32),
        pltpu.VMEM((WE, 32), jnp.float32),
        pltpu.VMEM((WE, 32), jnp.float32),
        pltpu.VMEM((WE, 32), jnp.float32),
        pltpu.VMEM_SHARED((N, 32), jnp.float32),
        pltpu.VMEM_SHARED((N, 32), jnp.float32),
        pltpu.VMEM_SHARED((N,), jnp.float32),
        pltpu.SemaphoreType.DMA,
        pltpu.SemaphoreType.DMA,
        pltpu.SemaphoreType.DMA,
        pltpu.SemaphoreType.DMA,
        pltpu.SemaphoreType.DMA,
        pltpu.SemaphoreType.DMA,
    ],
)


def _dense_body(degp_ref, s0_ref, s1_ref,
                wp0_ref, bp0_ref, g0_ref, be0_ref, wg0_ref, bg0_ref,
                wp1_ref, bp1_ref, g1_ref, be1_ref, wg1_ref, bg1_ref,
                xd0_ref, xd1_ref, deg_ref, dis_ref, disr_ref):
    deg = degp_ref[0] + degp_ref[1] + 1.0
    dis = lax.rsqrt(deg)
    deg_ref[...] = deg
    dis_ref[...] = dis
    disr_ref[...] = jnp.broadcast_to(dis, (N, 16))

    dn = (((1,), (1,)), ((), ()))

    def branch(s_ref, wp_ref, bp_ref, g_ref, be_ref, wg_ref, bg_ref, xd_ref):
        p = lax.dot_general(s_ref[...], wp_ref[...], dn,
                            preferred_element_type=jnp.float32)
        p = p + bp_ref[...]
        m = jnp.mean(p, axis=0, keepdims=True)
        v = jnp.mean((p - m) ** 2, axis=0, keepdims=True)
        n = (p - m) * lax.rsqrt(v + 1e-5) * g_ref[...] + be_ref[...]
        xin = jnp.maximum(n, 0.0)
        xs = lax.dot_general(xin, wg_ref[...], dn,
                             preferred_element_type=jnp.float32)
        xd_ref[...] = (xs + bg_ref[...]) * dis

    branch(s0_ref, wp0_ref, bp0_ref, g0_ref, be0_ref, wg0_ref, bg0_ref,
           xd0_ref)
    branch(s1_ref, wp1_ref, bp1_ref, g1_ref, be1_ref, wg1_ref, bg1_ref,
           xd1_ref)


_dense = pl.pallas_call(
    _dense_body,
    out_shape=(
        jax.ShapeDtypeStruct((N, D), jnp.float32),
        jax.ShapeDtypeStruct((N, D), jnp.float32),
        jax.ShapeDtypeStruct((N, 1), jnp.float32),
        jax.ShapeDtypeStruct((N, 1), jnp.float32),
        jax.ShapeDtypeStruct((N, 16), jnp.float32),
    ),
)


_RB = 2000
_NB = N // _RB


def _final_body(agg_ref, xd0_ref, xd1_ref, deg_ref, dis_ref, root_ref,
                g_ref, be_ref, out_ref, ys_ref, sums_ref, sqs_ref):
    ph = pl.program_id(0)
    rb = pl.program_id(1)

    @pl.when(jnp.logical_and(ph == 0, rb == 0))
    def _():
        sums_ref[...] = jnp.zeros_like(sums_ref)
        sqs_ref[...] = jnp.zeros_like(sqs_ref)

    rows = pl.ds(rb * _RB, _RB)

    @pl.when(ph == 0)
    def _():
        deg = deg_ref[...]
        sdeg = jnp.sqrt(deg)
        dis = dis_ref[...]
        for b in range(2):
            agg = agg_ref[b, 0] + agg_ref[b, 1]
            xd = xd0_ref[...] if b == 0 else xd1_ref[...]
            x = xd * sdeg
            self_t = jnp.maximum(x + root_ref[b], 0.0) / deg
            y = dis * agg + self_t
            ys_ref[b, rows, :] = y
            sums_ref[b, :] += jnp.sum(y, axis=0)
            sqs_ref[b, :] += jnp.sum(y * y, axis=0)
        out_ref[...] = jnp.zeros_like(out_ref)

    @pl.when(ph == 1)
    def _():
        acc = jnp.zeros((_RB, D), jnp.float32)
        for b in range(2):
            mean = sums_ref[b, :] / N
            var = sqs_ref[b, :] / N - mean * mean
            y = ys_ref[b, rows, :]
            st = (y - mean) * lax.rsqrt(var + 1e-5) * g_ref[b] + be_ref[b]
            acc = acc + st
        out_ref[...] = acc


_final = pl.pallas_call(
    _final_body,
    grid=(2, _NB),
    in_specs=[
        pl.BlockSpec((2, NC, _RB, D), lambda ph, rb: (0, 0, rb * (1 - ph), 0)),
        pl.BlockSpec((_RB, D), lambda ph, rb: (rb * (1 - ph), 0)),
        pl.BlockSpec((_RB, D), lambda ph, rb: (rb * (1 - ph), 0)),
        pl.BlockSpec((_RB, 1), lambda ph, rb: (rb, 0)),
        pl.BlockSpec((_RB, 1), lambda ph, rb: (rb, 0)),
        pl.BlockSpec((2, 1, D), lambda ph, rb: (0, 0, 0)),
        pl.BlockSpec((2, D), lambda ph, rb: (0, 0)),
        pl.BlockSpec((2, D), lambda ph, rb: (0, 0)),
    ],
    out_specs=pl.BlockSpec((_RB, D), lambda ph, rb: (rb, 0)),
    out_shape=jax.ShapeDtypeStruct((N, D), jnp.float32),
    scratch_shapes=[
        pltpu.VMEM((2, N, D), jnp.float32),
        pltpu.VMEM((2, D), jnp.float32),
        pltpu.VMEM((2, D), jnp.float32),
    ],
)


def kernel(s0, s1, edge_index, edge_attr, in_degree, out_degree, mat, batch,
           max_node, W_pre0, b_pre0, g_pre0, be_pre0, W_pre1, b_pre1, g_pre1,
           be_pre1, W_gcn0, b_gcn0, root0, g_op0, be_op0, W_gcn1, b_gcn1,
           root1, g_op1, be_op1):
    row = edge_index[0]
    col = edge_index[1]
    zeros_n = jnp.zeros((N,), jnp.float32)

    degp = _hist(row, zeros_n)
    xd0, xd1, deg, dis, disr = _dense(
        degp.reshape(NC, N, 1), s0, s1,
        W_pre0, b_pre0, g_pre0, be_pre0, W_gcn0, b_gcn0,
        W_pre1, b_pre1, g_pre1, be_pre1, W_gcn1, b_gcn1)

    splat = jnp.broadcast_to(
        jnp.arange(WE, dtype=jnp.int32)[:, None], (WE, 16))
    agg = _edge(row, col, edge_attr, xd0, xd1, dis.reshape(N), splat)

    root = jnp.stack([root0, root1])
    g = jnp.stack([g_op0, g_op1])
    be = jnp.stack([be_op0, be_op1])
    return _final(agg, xd0, xd1, deg, dis, root, g, be)
